# baseline (speedup 1.0000x reference)
import functools

import jax
import jax.numpy as jnp
from jax import lax
from jax.experimental import pallas as pl
from jax.experimental.pallas import tpu as pltpu
from jax.experimental.pallas import tpu_sc as plsc

GRID = 64
STRIDE = 2
OG = GRID // STRIDE
BATCH = 4
N = 100000
C = 256
NUM_SEGMENTS = BATCH * OG * OG * OG

NC, NS, L = 2, 16, 16
NW = NC * NS
PTS_W = 3136
NP = PTS_W * NW
PAD = NP - N

CHUNK = 256
PASSES = NUM_SEGMENTS // (NW * CHUNK)
KCH = 2048
NKC = NP // KCH
LCAP = 6144
DUMMY_OFF = CHUNK * C
NEG = float("-inf")

_mesh = plsc.VectorSubcoreMesh(core_axis_name="c", subcore_axis_name="s")


@functools.partial(
    pl.kernel,
    out_type=jax.ShapeDtypeStruct((NP,), jnp.int32),
    mesh=_mesh,
    compiler_params=pltpu.CompilerParams(needs_layout_passes=False),
    scratch_types=[
        pltpu.VMEM((PTS_W,), jnp.int32),
        pltpu.VMEM((PTS_W,), jnp.int32),
        pltpu.VMEM((PTS_W,), jnp.int32),
        pltpu.VMEM((PTS_W,), jnp.int32),
        pltpu.VMEM((PTS_W,), jnp.int32),
    ],
)
def _keys_kernel(xs_hbm, ys_hbm, zs_hbm, batch_hbm, keys_hbm,
                 xslab, yslab, zslab, bslab, kslab):
    wid = lax.axis_index("s") * NC + lax.axis_index("c")
    base = wid * PTS_W
    pltpu.sync_copy(xs_hbm.at[pl.ds(base, PTS_W)], xslab)
    pltpu.sync_copy(ys_hbm.at[pl.ds(base, PTS_W)], yslab)
    pltpu.sync_copy(zs_hbm.at[pl.ds(base, PTS_W)], zslab)
    pltpu.sync_copy(batch_hbm.at[pl.ds(base, PTS_W)], bslab)

    def body(i, carry):
        x = xslab[pl.ds(i * L, L)]
        y = yslab[pl.ds(i * L, L)]
        z = zslab[pl.ds(i * L, L)]
        b = bslab[pl.ds(i * L, L)]
        key = (b * (OG * OG * OG) + (x >> 1) * (OG * OG)
               + (y >> 1) * OG + (z >> 1))
        kslab[pl.ds(i * L, L)] = key
        return carry

    lax.fori_loop(0, PTS_W // L, body, jnp.int32(0))
    pltpu.sync_copy(kslab, keys_hbm.at[pl.ds(base, PTS_W)])


@functools.partial(
    pl.kernel,
    out_type=jax.ShapeDtypeStruct((NUM_SEGMENTS * C,), jnp.float32),
    mesh=_mesh,
    compiler_params=pltpu.CompilerParams(needs_layout_passes=False),
    scratch_types=[
        pltpu.VMEM((KCH,), jnp.int32),
        pltpu.VMEM((LCAP + 2 * L,), jnp.int32),
        pltpu.VMEM((LCAP + 2 * L,), jnp.int32),
        pltpu.VMEM((L, C), jnp.float32),
        pltpu.VMEM(((CHUNK + 1) * C,), jnp.float32),
        pltpu.SemaphoreType.DMA,
    ],
)
def _pool_kernel(feats_hbm, keys_hbm, out_hbm, kbuf, soff, spid, rowbuf, acc, sem):
    wid = lax.axis_index("s") * NC + lax.axis_index("c")
    lane = lax.iota(jnp.int32, L)

    def drain(cnt):
        soff[pl.ds(cnt, L)] = jnp.full((L,), DUMMY_OFF, jnp.int32)
        spid[pl.ds(cnt, L)] = jnp.zeros((L,), jnp.int32)
        nb = (cnt + (L - 1)) // L

        def gbody(g, carry):
            idxv = spid[pl.ds(g * L, L)]
            pltpu.async_copy(feats_hbm.at[idxv], rowbuf, sem).wait()

            def pbody(i, c2):
                off = soff[pl.ds(g * L + i, L)][0]
                for j in range(C // L):
                    a = acc[pl.ds(off + j * L, L)]
                    r = rowbuf[i, pl.ds(j * L, L)]
                    acc[pl.ds(off + j * L, L)] = jnp.maximum(a, r)
                return c2

            lax.fori_loop(0, L, pbody, jnp.int32(0))
            return carry

        lax.fori_loop(0, nb, gbody, jnp.int32(0))

    def run_pass(p, carry):
        base_row = (p * NW + wid) * CHUNK

        def ibody(v, c):
            acc[pl.ds(v * L, L)] = jnp.full((L,), NEG, jnp.float32)
            return c

        lax.fori_loop(0, CHUNK * C // L, ibody, jnp.int32(0))

        def kc_body(kc, cnt):
            pltpu.sync_copy(keys_hbm.at[pl.ds(kc * KCH, KCH)], kbuf)

            def scan_body(v, c):
                k = kbuf[pl.ds(v * L, L)]
                rel = k - base_row
                m = (rel >= 0) & (rel < CHUNK)
                pid = kc * KCH + v * L + lane
                plsc.store_compressed(soff.at[pl.ds(c, L)], rel * C, mask=m)
                plsc.store_compressed(spid.at[pl.ds(c, L)], pid, mask=m)
                return c + jnp.sum(m.astype(jnp.int32))

            cnt = lax.fori_loop(0, KCH // L, scan_body, cnt)

            def do_drain(c):
                drain(c)
                return jnp.int32(0)

            return lax.cond(cnt >= LCAP - KCH, do_drain, lambda c: c, cnt)

        cnt = lax.fori_loop(0, NKC, kc_body, jnp.int32(0))
        drain(cnt)

        def fbody(v, c):
            a = acc[pl.ds(v * L, L)]
            acc[pl.ds(v * L, L)] = jnp.where(a == NEG, jnp.float32(0.0), a)
            return c

        lax.fori_loop(0, CHUNK * C // L, fbody, jnp.int32(0))
        pltpu.sync_copy(acc.at[pl.ds(0, CHUNK * C)],
                        out_hbm.at[pl.ds(base_row * C, CHUNK * C)])
        return carry

    lax.fori_loop(0, PASSES, run_pass, jnp.int32(0))


def kernel(feats, coords, batch_idx):
    zpad = jnp.zeros((PAD,), jnp.int32)
    xs = jnp.concatenate([coords[:, 0], zpad])
    ys = jnp.concatenate([coords[:, 1], zpad])
    zs = jnp.concatenate([coords[:, 2], zpad])
    batch_flat = jnp.concatenate(
        [batch_idx.reshape(-1).astype(jnp.int32), jnp.full((PAD,), BATCH, jnp.int32)])
    keys = _keys_kernel(xs, ys, zs, batch_flat)
    out = _pool_kernel(feats, keys)
    return out.reshape(NUM_SEGMENTS, C)


# speedup vs baseline: 2.1065x; 2.1065x over previous
import functools

import jax
import jax.numpy as jnp
from jax import lax
from jax.experimental import pallas as pl
from jax.experimental.pallas import tpu as pltpu
from jax.experimental.pallas import tpu_sc as plsc

GRID = 64
STRIDE = 2
OG = GRID // STRIDE
BATCH = 4
N = 100000
C = 256
NUM_SEGMENTS = BATCH * OG * OG * OG

NC, NS, L = 2, 16, 16
NW = NC * NS
PTS_W = 3136
NP = PTS_W * NW
PAD = NP - N

OWN_ROWS = NUM_SEGMENTS // NW
CHUNK = 256
PASSES = OWN_ROWS // CHUNK

BLK = 128
BLKW = 2 * BLK
NBLK_W = PTS_W // BLK
RCAP = ((PTS_W + BLK - 1) // BLK) * BLK
RCAP2 = 2 * RCAP
SELCAP = RCAP + 2 * L
BMAX = NW * (RCAP // BLK) + L
SCAP = 64
DRAIN_T = 2048
LCAP = DRAIN_T + BLK + 2 * L
G = 4
DUMMY_OFF = CHUNK * C
NEG = float("-inf")

_mesh = plsc.VectorSubcoreMesh(core_axis_name="c", subcore_axis_name="s")
_params = pltpu.CompilerParams(needs_layout_passes=False)


@functools.partial(
    pl.kernel,
    out_type=(jax.ShapeDtypeStruct((NW * NW * RCAP2,), jnp.int32),
              jax.ShapeDtypeStruct((NW * NW,), jnp.int32)),
    mesh=_mesh,
    compiler_params=_params,
    scratch_types=[
        pltpu.VMEM((PTS_W,), jnp.int32),
        pltpu.VMEM((PTS_W,), jnp.int32),
        pltpu.VMEM((PTS_W,), jnp.int32),
        pltpu.VMEM((PTS_W,), jnp.int32),
        pltpu.VMEM((PTS_W,), jnp.int32),
        pltpu.VMEM((SELCAP,), jnp.int32),
        pltpu.VMEM((SELCAP,), jnp.int32),
        pltpu.VMEM((SELCAP,), jnp.int32),
        pltpu.VMEM((SELCAP,), jnp.int32),
        pltpu.VMEM((32,), jnp.int32),
        pltpu.SemaphoreType.DMA,
        pltpu.SemaphoreType.DMA,
    ],
)
def _bin_kernel(xs_hbm, ys_hbm, zs_hbm, batch_hbm, bins_hbm, counts_hbm,
                xslab, yslab, zslab, bslab, kslab,
                selk0, selp0, selk1, selp1, cbuf, fsem0, fsem1):
    wid = lax.axis_index("s") * NC + lax.axis_index("c")
    wbase = wid * PTS_W
    lane = lax.iota(jnp.int32, L)

    pltpu.sync_copy(xs_hbm.at[pl.ds(wbase, PTS_W)], xslab)
    pltpu.sync_copy(ys_hbm.at[pl.ds(wbase, PTS_W)], yslab)
    pltpu.sync_copy(zs_hbm.at[pl.ds(wbase, PTS_W)], zslab)
    pltpu.sync_copy(batch_hbm.at[pl.ds(wbase, PTS_W)], bslab)

    def kbody(i, carry):
        x = xslab[pl.ds(i * L, L)]
        y = yslab[pl.ds(i * L, L)]
        z = zslab[pl.ds(i * L, L)]
        b = bslab[pl.ds(i * L, L)]
        key = (b * (OG * OG * OG) + (x >> 1) * (OG * OG)
               + (y >> 1) * OG + (z >> 1))
        kslab[pl.ds(i * L, L)] = key
        return carry

    lax.fori_loop(0, PTS_W // L, kbody, jnp.int32(0))

    sel = [(selk0, selp0, fsem0), (selk1, selp1, fsem1)]
    cnts_py = []
    for o in range(NW):
        selk, selp, fsem = sel[o % 2]
        if o >= 2:
            prev_cnt = cnts_py[o - 2]
            nblk_prev = (prev_cnt + (BLK - 1)) // BLK

            def wbody(i, carry, _selk=selk, _selp=selp, _fsem=fsem):
                pltpu.make_async_copy(
                    bins_hbm.at[pl.ds(0, BLK)], _selk.at[pl.ds(0, BLK)],
                    _fsem).wait()
                pltpu.make_async_copy(
                    bins_hbm.at[pl.ds(0, BLK)], _selp.at[pl.ds(0, BLK)],
                    _fsem).wait()
                return carry

            lax.fori_loop(0, nblk_prev, wbody, jnp.int32(0))

        def sbody(v, cnt, _o=o, _selk=selk, _selp=selp):
            k = kslab[pl.ds(v * L, L)]
            m = (k >> 12) == _o
            pid = wbase + v * L + lane
            plsc.store_compressed(_selk.at[pl.ds(cnt, L)], k, mask=m)
            plsc.store_compressed(_selp.at[pl.ds(cnt, L)], pid, mask=m)
            return cnt + jnp.sum(m.astype(jnp.int32))

        cnt = lax.fori_loop(0, PTS_W // L, sbody, jnp.int32(0))
        cnts_py.append(cnt)

        base_off = (wid * NW + o) * RCAP2
        nblk = (cnt + (BLK - 1)) // BLK

        def fbody(blk, carry, _selk=selk, _selp=selp, _fsem=fsem,
                  _base=base_off):
            off = _base + blk * BLKW
            pltpu.async_copy(_selk.at[pl.ds(blk * BLK, BLK)],
                             bins_hbm.at[pl.ds(off, BLK)], _fsem)
            pltpu.async_copy(_selp.at[pl.ds(blk * BLK, BLK)],
                             bins_hbm.at[pl.ds(off + BLK, BLK)], _fsem)
            return carry

        lax.fori_loop(0, nblk, fbody, jnp.int32(0))

    for o in (NW - 2, NW - 1):
        selk, selp, fsem = sel[o % 2]
        nblk_prev = (cnts_py[o] + (BLK - 1)) // BLK

        def wbody(i, carry, _selk=selk, _selp=selp, _fsem=fsem):
            pltpu.make_async_copy(bins_hbm.at[pl.ds(0, BLK)],
                                  _selk.at[pl.ds(0, BLK)], _fsem).wait()
            pltpu.make_async_copy(bins_hbm.at[pl.ds(0, BLK)],
                                  _selp.at[pl.ds(0, BLK)], _fsem).wait()
            return carry

        lax.fori_loop(0, nblk_prev, wbody, jnp.int32(0))

    v0 = jnp.zeros((L,), jnp.int32)
    v1 = jnp.zeros((L,), jnp.int32)
    for o in range(L):
        v0 = jnp.where(lane == o, cnts_py[o], v0)
        v1 = jnp.where(lane == o, cnts_py[L + o], v1)
    cbuf[pl.ds(0, L)] = v0
    cbuf[pl.ds(L, L)] = v1
    pltpu.sync_copy(cbuf, counts_hbm.at[pl.ds(wid * NW, NW)])


@functools.partial(
    pl.kernel,
    out_type=jax.ShapeDtypeStruct((NUM_SEGMENTS * C,), jnp.float32),
    mesh=_mesh,
    compiler_params=_params,
    scratch_types=[
        pltpu.VMEM((NW * NW + L,), jnp.int32),
        pltpu.VMEM((BMAX + L,), jnp.int32),
        pltpu.VMEM((BMAX + L,), jnp.int32),
        pltpu.VMEM((SCAP * BLKW,), jnp.int32),
        pltpu.VMEM((LCAP + 2 * L,), jnp.int32),
        pltpu.VMEM((LCAP + 2 * L,), jnp.int32),
        pltpu.VMEM((G, L, C), jnp.float32),
        pltpu.VMEM(((CHUNK + 1) * C,), jnp.float32),
        pltpu.SemaphoreType.DMA,
        pltpu.SemaphoreType.DMA,
    ],
)
def _pool_kernel(feats_hbm, bins_hbm, counts_hbm, out_hbm,
                 cvm, blkoff, blkval, stage, soff, spid, rowbuf, acc,
                 ksem, gsem):
    wid = lax.axis_index("s") * NC + lax.axis_index("c")
    lane = lax.iota(jnp.int32, L)

    pltpu.sync_copy(counts_hbm, cvm.at[pl.ds(0, NW * NW)])

    nb_total = jnp.int32(0)
    for wp in range(NW):
        c = cvm[pl.ds(wp * NW + wid, L)][0]
        nblk = (c + (BLK - 1)) // BLK
        src0 = (wp * NW + wid) * RCAP2

        def put(blk, pos, _src0=src0, _c=c):
            blkoff[pl.ds(pos, L)] = jnp.zeros((L,), jnp.int32) + (_src0 + blk * BLKW)
            blkval[pl.ds(pos, L)] = jnp.zeros((L,), jnp.int32) + jnp.minimum(_c - blk * BLK, BLK)
            return pos + 1

        nb_total = lax.fori_loop(0, nblk, put, nb_total)

    def drain(cnt):
        soff[pl.ds(cnt, L)] = jnp.full((L,), DUMMY_OFF, jnp.int32)
        spid[pl.ds(cnt, L)] = jnp.zeros((L,), jnp.int32)
        nb = (cnt + (L - 1)) // L

        def super_g(sg, carry):
            gstart = sg * G
            ng = jnp.minimum(nb - gstart, G)

            def fire2(i, c2):
                idxv = spid[pl.ds((gstart + i) * L, L)]
                pltpu.async_copy(feats_hbm.at[idxv], rowbuf.at[i], gsem)
                return c2

            lax.fori_loop(0, ng, fire2, jnp.int32(0))

            def drng(i, c2):
                idxv = spid[pl.ds((gstart + i) * L, L)]
                pltpu.make_async_copy(feats_hbm.at[idxv], rowbuf.at[i],
                                      gsem).wait()
                return c2

            lax.fori_loop(0, ng, drng, jnp.int32(0))

            def gbody(i, c2):
                def pbody(ii, c3):
                    off = soff[pl.ds((gstart + i) * L + ii, L)][0]
                    for jj in range(C // L):
                        a = acc[pl.ds(off + jj * L, L)]
                        r = rowbuf[i, ii, pl.ds(jj * L, L)]
                        acc[pl.ds(off + jj * L, L)] = jnp.maximum(a, r)
                    return c3

                lax.fori_loop(0, L, pbody, jnp.int32(0))
                return c2

            lax.fori_loop(0, ng, gbody, jnp.int32(0))
            return carry

        nsg = (nb + (G - 1)) // G
        lax.fori_loop(0, nsg, super_g, jnp.int32(0))

    def run_pass(p, carry):
        base_row = wid * OWN_ROWS + p * CHUNK

        def ibody(v, c):
            acc[pl.ds(v * L, L)] = jnp.full((L,), NEG, jnp.float32)
            return c

        lax.fori_loop(0, CHUNK * C // L, ibody, jnp.int32(0))

        def super_body(s, cnt):
            start = s * SCAP
            nb_s = jnp.minimum(nb_total - start, SCAP)

            def fire(i, c2):
                off = pl.multiple_of(blkoff[pl.ds(start + i, L)][0], BLKW)
                pltpu.async_copy(bins_hbm.at[pl.ds(off, BLKW)],
                                 stage.at[pl.ds(i * BLKW, BLKW)], ksem)
                return c2

            lax.fori_loop(0, nb_s, fire, jnp.int32(0))

            def drk(i, c2):
                pltpu.make_async_copy(bins_hbm.at[pl.ds(0, BLKW)],
                                      stage.at[pl.ds(0, BLKW)], ksem).wait()
                return c2

            lax.fori_loop(0, nb_s, drk, jnp.int32(0))

            def block_body(i, cnt2):
                val = blkval[pl.ds(start + i, L)][0]
                sbase = i * BLKW

                def scan_body(j, cnt3):
                    k = stage[pl.ds(sbase + j * L, L)]
                    pidv = stage[pl.ds(sbase + BLK + j * L, L)]
                    rel = k - base_row
                    m = ((j * L + lane) < val) & (rel >= 0) & (rel < CHUNK)
                    plsc.store_compressed(soff.at[pl.ds(cnt3, L)], rel * C,
                                          mask=m)
                    plsc.store_compressed(spid.at[pl.ds(cnt3, L)], pidv,
                                          mask=m)
                    return cnt3 + jnp.sum(m.astype(jnp.int32))

                cnt2 = lax.fori_loop(0, BLK // L, scan_body, cnt2)

                def do_drain(c):
                    drain(c)
                    return jnp.int32(0)

                return lax.cond(cnt2 >= DRAIN_T, do_drain, lambda c: c, cnt2)

            return lax.fori_loop(0, nb_s, block_body, cnt)

        nsuper = (nb_total + (SCAP - 1)) // SCAP
        cnt = lax.fori_loop(0, nsuper, super_body, jnp.int32(0))
        drain(cnt)

        def fbody(v, c):
            a = acc[pl.ds(v * L, L)]
            acc[pl.ds(v * L, L)] = jnp.where(a == NEG, jnp.float32(0.0), a)
            return c

        lax.fori_loop(0, CHUNK * C // L, fbody, jnp.int32(0))
        pltpu.sync_copy(acc.at[pl.ds(0, CHUNK * C)],
                        out_hbm.at[pl.ds(base_row * C, CHUNK * C)])
        return carry

    lax.fori_loop(0, PASSES, run_pass, jnp.int32(0))


def kernel(feats, coords, batch_idx):
    zpad = jnp.zeros((PAD,), jnp.int32)
    xs = jnp.concatenate([coords[:, 0], zpad])
    ys = jnp.concatenate([coords[:, 1], zpad])
    zs = jnp.concatenate([coords[:, 2], zpad])
    batch_flat = jnp.concatenate(
        [batch_idx.reshape(-1).astype(jnp.int32),
         jnp.full((PAD,), BATCH, jnp.int32)])
    bins, counts = _bin_kernel(xs, ys, zs, batch_flat)
    out = _pool_kernel(feats, bins, counts)
    return out.reshape(NUM_SEGMENTS, C)


# speedup vs baseline: 3.4278x; 1.6272x over previous
import functools

import jax
import jax.numpy as jnp
from jax import lax
from jax.experimental import pallas as pl
from jax.experimental.pallas import tpu as pltpu
from jax.experimental.pallas import tpu_sc as plsc

GRID = 64
STRIDE = 2
OG = GRID // STRIDE
BATCH = 4
N = 100000
C = 256
NUM_SEGMENTS = BATCH * OG * OG * OG

NC, NS, L = 2, 16, 16
NW = NC * NS
PTS_W = 3136
NP = PTS_W * NW
PAD = NP - N

OWN_ROWS = NUM_SEGMENTS // NW
CHUNK = 256
PASSES = OWN_ROWS // CHUNK

BLK = 128
BLKW = 2 * BLK
NBLK_W = PTS_W // BLK
RCAP = ((PTS_W + BLK - 1) // BLK) * BLK
RCAP2 = 2 * RCAP
SELCAP = RCAP + 2 * L
BMAX = NW * (RCAP // BLK) + L
SCAP = 64
DRAIN_T = 2048
LCAP = DRAIN_T + BLK + 2 * L
G = 4
DUMMY_OFF = CHUNK * C
NEG = float("-inf")

_mesh = plsc.VectorSubcoreMesh(core_axis_name="c", subcore_axis_name="s")
_params = pltpu.CompilerParams(needs_layout_passes=False)


@functools.partial(
    pl.kernel,
    out_type=(jax.ShapeDtypeStruct((NW * NW * RCAP2,), jnp.int32),
              jax.ShapeDtypeStruct((NW * NW,), jnp.int32)),
    mesh=_mesh,
    compiler_params=_params,
    scratch_types=[
        pltpu.VMEM((PTS_W,), jnp.int32),
        pltpu.VMEM((PTS_W,), jnp.int32),
        pltpu.VMEM((PTS_W,), jnp.int32),
        pltpu.VMEM((PTS_W,), jnp.int32),
        pltpu.VMEM((PTS_W,), jnp.int32),
        pltpu.VMEM((SELCAP,), jnp.int32),
        pltpu.VMEM((SELCAP,), jnp.int32),
        pltpu.VMEM((SELCAP,), jnp.int32),
        pltpu.VMEM((SELCAP,), jnp.int32),
        pltpu.VMEM((32,), jnp.int32),
        pltpu.SemaphoreType.DMA,
        pltpu.SemaphoreType.DMA,
    ],
)
def _bin_kernel(xs_hbm, ys_hbm, zs_hbm, batch_hbm, bins_hbm, counts_hbm,
                xslab, yslab, zslab, bslab, kslab,
                selk0, selp0, selk1, selp1, cbuf, fsem0, fsem1):
    wid = lax.axis_index("s") * NC + lax.axis_index("c")
    wbase = wid * PTS_W
    lane = lax.iota(jnp.int32, L)

    pltpu.sync_copy(xs_hbm.at[pl.ds(wbase, PTS_W)], xslab)
    pltpu.sync_copy(ys_hbm.at[pl.ds(wbase, PTS_W)], yslab)
    pltpu.sync_copy(zs_hbm.at[pl.ds(wbase, PTS_W)], zslab)
    pltpu.sync_copy(batch_hbm.at[pl.ds(wbase, PTS_W)], bslab)

    def kbody(i, carry):
        x = xslab[pl.ds(i * L, L)]
        y = yslab[pl.ds(i * L, L)]
        z = zslab[pl.ds(i * L, L)]
        b = bslab[pl.ds(i * L, L)]
        key = (b * (OG * OG * OG) + (x >> 1) * (OG * OG)
               + (y >> 1) * OG + (z >> 1))
        kslab[pl.ds(i * L, L)] = key
        return carry

    lax.fori_loop(0, PTS_W // L, kbody, jnp.int32(0))

    sel = [(selk0, selp0, fsem0), (selk1, selp1, fsem1)]
    cnts_py = []
    for o in range(NW):
        selk, selp, fsem = sel[o % 2]
        if o >= 2:
            prev_cnt = cnts_py[o - 2]
            nblk_prev = (prev_cnt + (BLK - 1)) // BLK

            def wbody(i, carry, _selk=selk, _selp=selp, _fsem=fsem):
                pltpu.make_async_copy(
                    bins_hbm.at[pl.ds(0, BLK)], _selk.at[pl.ds(0, BLK)],
                    _fsem).wait()
                pltpu.make_async_copy(
                    bins_hbm.at[pl.ds(0, BLK)], _selp.at[pl.ds(0, BLK)],
                    _fsem).wait()
                return carry

            lax.fori_loop(0, nblk_prev, wbody, jnp.int32(0))

        def sbody(v, cnt, _o=o, _selk=selk, _selp=selp):
            k = kslab[pl.ds(v * L, L)]
            m = (k >> 12) == _o
            pid = wbase + v * L + lane
            plsc.store_compressed(_selk.at[pl.ds(cnt, L)], k, mask=m)
            plsc.store_compressed(_selp.at[pl.ds(cnt, L)], pid, mask=m)
            return cnt + jnp.sum(m.astype(jnp.int32))

        cnt = lax.fori_loop(0, PTS_W // L, sbody, jnp.int32(0))
        cnts_py.append(cnt)

        base_off = (wid * NW + o) * RCAP2
        nblk = (cnt + (BLK - 1)) // BLK

        def fbody(blk, carry, _selk=selk, _selp=selp, _fsem=fsem,
                  _base=base_off):
            off = _base + blk * BLKW
            pltpu.async_copy(_selk.at[pl.ds(blk * BLK, BLK)],
                             bins_hbm.at[pl.ds(off, BLK)], _fsem)
            pltpu.async_copy(_selp.at[pl.ds(blk * BLK, BLK)],
                             bins_hbm.at[pl.ds(off + BLK, BLK)], _fsem)
            return carry

        lax.fori_loop(0, nblk, fbody, jnp.int32(0))

    for o in (NW - 2, NW - 1):
        selk, selp, fsem = sel[o % 2]
        nblk_prev = (cnts_py[o] + (BLK - 1)) // BLK

        def wbody(i, carry, _selk=selk, _selp=selp, _fsem=fsem):
            pltpu.make_async_copy(bins_hbm.at[pl.ds(0, BLK)],
                                  _selk.at[pl.ds(0, BLK)], _fsem).wait()
            pltpu.make_async_copy(bins_hbm.at[pl.ds(0, BLK)],
                                  _selp.at[pl.ds(0, BLK)], _fsem).wait()
            return carry

        lax.fori_loop(0, nblk_prev, wbody, jnp.int32(0))

    v0 = jnp.zeros((L,), jnp.int32)
    v1 = jnp.zeros((L,), jnp.int32)
    for o in range(L):
        v0 = jnp.where(lane == o, cnts_py[o], v0)
        v1 = jnp.where(lane == o, cnts_py[L + o], v1)
    cbuf[pl.ds(0, L)] = v0
    cbuf[pl.ds(L, L)] = v1
    pltpu.sync_copy(cbuf, counts_hbm.at[pl.ds(wid * NW, NW)])


@functools.partial(
    pl.kernel,
    out_type=jax.ShapeDtypeStruct((NUM_SEGMENTS * C,), jnp.float32),
    mesh=_mesh,
    compiler_params=_params,
    scratch_types=[
        pltpu.VMEM((NW * NW + L,), jnp.int32),
        pltpu.VMEM((BMAX + L,), jnp.int32),
        pltpu.VMEM((BMAX + L,), jnp.int32),
        pltpu.VMEM((SCAP * BLKW,), jnp.int32),
        pltpu.VMEM((LCAP + 2 * L,), jnp.int32),
        pltpu.VMEM((LCAP + 2 * L,), jnp.int32),
        pltpu.VMEM((G, L, C), jnp.float32),
        pltpu.VMEM(((CHUNK + 1) * C,), jnp.float32),
        pltpu.SemaphoreType.DMA,
        pltpu.SemaphoreType.DMA,
    ],
)
def _pool_kernel(feats_hbm, bins_hbm, counts_hbm, out_hbm,
                 cvm, blkoff, blkval, stage, soff, spid, rowbuf, acc,
                 ksem, gsem):
    wid = lax.axis_index("s") * NC + lax.axis_index("c")
    lane = lax.iota(jnp.int32, L)

    pltpu.sync_copy(counts_hbm, cvm.at[pl.ds(0, NW * NW)])

    nb_total = jnp.int32(0)
    for wp in range(NW):
        c = cvm[pl.ds(wp * NW + wid, L)][0]
        nblk = (c + (BLK - 1)) // BLK
        src0 = (wp * NW + wid) * RCAP2

        def put(blk, pos, _src0=src0, _c=c):
            blkoff[pl.ds(pos, L)] = jnp.zeros((L,), jnp.int32) + (_src0 + blk * BLKW)
            blkval[pl.ds(pos, L)] = jnp.zeros((L,), jnp.int32) + jnp.minimum(_c - blk * BLK, BLK)
            return pos + 1

        nb_total = lax.fori_loop(0, nblk, put, nb_total)

    def drain(cnt):
        soff[pl.ds(cnt, L)] = jnp.full((L,), DUMMY_OFF, jnp.int32)
        spid[pl.ds(cnt, L)] = jnp.zeros((L,), jnp.int32)
        nb = (cnt + (L - 1)) // L

        def super_g(sg, carry):
            gstart = sg * G
            ng = jnp.minimum(nb - gstart, G)

            def fire2(i, c2):
                idxv = spid[pl.ds((gstart + i) * L, L)]
                pltpu.async_copy(feats_hbm.at[idxv], rowbuf.at[i], gsem)
                return c2

            lax.fori_loop(0, ng, fire2, jnp.int32(0))

            def drng(i, c2):
                idxv = spid[pl.ds((gstart + i) * L, L)]
                pltpu.make_async_copy(feats_hbm.at[idxv], rowbuf.at[i],
                                      gsem).wait()
                return c2

            lax.fori_loop(0, ng, drng, jnp.int32(0))

            def gbody(i, c2):
                def pbody(ii, c3):
                    off = soff[pl.ds((gstart + i) * L + ii, L)][0]
                    for jj in range(C // L):
                        a = acc[pl.ds(off + jj * L, L)]
                        r = rowbuf[i, ii, pl.ds(jj * L, L)]
                        acc[pl.ds(off + jj * L, L)] = jnp.maximum(a, r)
                    return c3

                lax.fori_loop(0, L, pbody, jnp.int32(0))
                return c2

            lax.fori_loop(0, ng, gbody, jnp.int32(0))
            return carry

        nsg = (nb + (G - 1)) // G
        lax.fori_loop(0, nsg, super_g, jnp.int32(0))

    def run_pass(p, carry):
        base_row = wid * OWN_ROWS + p * CHUNK

        neg = jnp.full((L,), NEG, jnp.float32)

        def ibody(v, c):
            for u in range(16):
                acc[pl.ds(v * (16 * L) + u * L, L)] = neg
            return c

        lax.fori_loop(0, CHUNK * C // (16 * L), ibody, jnp.int32(0))

        def super_body(s, cnt):
            start = s * SCAP
            nb_s = jnp.minimum(nb_total - start, SCAP)

            def fire(i, c2):
                off = pl.multiple_of(blkoff[pl.ds(start + i, L)][0], BLKW)
                pltpu.async_copy(bins_hbm.at[pl.ds(off, BLKW)],
                                 stage.at[pl.ds(i * BLKW, BLKW)], ksem)
                return c2

            lax.fori_loop(0, nb_s, fire, jnp.int32(0))

            def drk(i, c2):
                pltpu.make_async_copy(bins_hbm.at[pl.ds(0, BLKW)],
                                      stage.at[pl.ds(0, BLKW)], ksem).wait()
                return c2

            lax.fori_loop(0, nb_s, drk, jnp.int32(0))

            def block_body(i, cnt2):
                val = blkval[pl.ds(start + i, L)][0]
                sbase = i * BLKW

                def scan_step(j, cnt3):
                    k = stage[pl.ds(sbase + j * L, L)]
                    pidv = stage[pl.ds(sbase + BLK + j * L, L)]
                    rel = k - base_row
                    m = ((j * L + lane) < val) & (rel >= 0) & (rel < CHUNK)
                    plsc.store_compressed(soff.at[pl.ds(cnt3, L)], rel * C,
                                          mask=m)
                    plsc.store_compressed(spid.at[pl.ds(cnt3, L)], pidv,
                                          mask=m)
                    return cnt3 + jnp.sum(m.astype(jnp.int32))

                for j in range(BLK // L):
                    cnt2 = scan_step(j, cnt2)

                def do_drain(c):
                    drain(c)
                    return jnp.int32(0)

                return lax.cond(cnt2 >= DRAIN_T, do_drain, lambda c: c, cnt2)

            return lax.fori_loop(0, nb_s, block_body, cnt)

        nsuper = (nb_total + (SCAP - 1)) // SCAP
        cnt = lax.fori_loop(0, nsuper, super_body, jnp.int32(0))
        drain(cnt)

        def fbody(v, c):
            for u in range(8):
                a = acc[pl.ds(v * (8 * L) + u * L, L)]
                acc[pl.ds(v * (8 * L) + u * L, L)] = jnp.where(
                    a == NEG, jnp.float32(0.0), a)
            return c

        lax.fori_loop(0, CHUNK * C // (8 * L), fbody, jnp.int32(0))
        pltpu.sync_copy(acc.at[pl.ds(0, CHUNK * C)],
                        out_hbm.at[pl.ds(base_row * C, CHUNK * C)])
        return carry

    lax.fori_loop(0, PASSES, run_pass, jnp.int32(0))


def kernel(feats, coords, batch_idx):
    zpad = jnp.zeros((PAD,), jnp.int32)
    xs = jnp.concatenate([coords[:, 0], zpad])
    ys = jnp.concatenate([coords[:, 1], zpad])
    zs = jnp.concatenate([coords[:, 2], zpad])
    batch_flat = jnp.concatenate(
        [batch_idx.reshape(-1).astype(jnp.int32),
         jnp.full((PAD,), BATCH, jnp.int32)])
    bins, counts = _bin_kernel(xs, ys, zs, batch_flat)
    out = _pool_kernel(feats, bins, counts)
    return out.reshape(NUM_SEGMENTS, C)


# speedup vs baseline: 3.5984x; 1.0498x over previous
import functools

import jax
import jax.numpy as jnp
from jax import lax
from jax.experimental import pallas as pl
from jax.experimental.pallas import tpu as pltpu
from jax.experimental.pallas import tpu_sc as plsc

GRID = 64
STRIDE = 2
OG = GRID // STRIDE
BATCH = 4
N = 100000
C = 256
NUM_SEGMENTS = BATCH * OG * OG * OG

NC, NS, L = 2, 16, 16
NW = NC * NS
PTS_W = 3136
NP = PTS_W * NW
PAD = NP - N

OWN_ROWS = NUM_SEGMENTS // NW
CHUNK = 256
PASSES = OWN_ROWS // CHUNK

BLK = 128
BLKW = 2 * BLK
NBLK_W = PTS_W // BLK
RCAP = ((PTS_W + BLK - 1) // BLK) * BLK
RCAP2 = 2 * RCAP
SELCAP = RCAP + 2 * L
BMAX = NW * (RCAP // BLK) + L
SCAP = 64
DRAIN_T = 2048
LCAP = DRAIN_T + BLK + 2 * L
G = 8
DUMMY_OFF = CHUNK * C
NEG = float("-inf")

_mesh = plsc.VectorSubcoreMesh(core_axis_name="c", subcore_axis_name="s")
_params = pltpu.CompilerParams(needs_layout_passes=False)


@functools.partial(
    pl.kernel,
    out_type=(jax.ShapeDtypeStruct((NW * NW * RCAP2,), jnp.int32),
              jax.ShapeDtypeStruct((NW * NW,), jnp.int32)),
    mesh=_mesh,
    compiler_params=_params,
    scratch_types=[
        pltpu.VMEM((PTS_W,), jnp.int32),
        pltpu.VMEM((PTS_W,), jnp.int32),
        pltpu.VMEM((PTS_W,), jnp.int32),
        pltpu.VMEM((PTS_W,), jnp.int32),
        pltpu.VMEM((PTS_W,), jnp.int32),
        pltpu.VMEM((SELCAP,), jnp.int32),
        pltpu.VMEM((SELCAP,), jnp.int32),
        pltpu.VMEM((SELCAP,), jnp.int32),
        pltpu.VMEM((SELCAP,), jnp.int32),
        pltpu.VMEM((32,), jnp.int32),
        pltpu.SemaphoreType.DMA,
        pltpu.SemaphoreType.DMA,
    ],
)
def _bin_kernel(xs_hbm, ys_hbm, zs_hbm, batch_hbm, bins_hbm, counts_hbm,
                xslab, yslab, zslab, bslab, kslab,
                selk0, selp0, selk1, selp1, cbuf, fsem0, fsem1):
    wid = lax.axis_index("s") * NC + lax.axis_index("c")
    wbase = wid * PTS_W
    lane = lax.iota(jnp.int32, L)

    pltpu.sync_copy(xs_hbm.at[pl.ds(wbase, PTS_W)], xslab)
    pltpu.sync_copy(ys_hbm.at[pl.ds(wbase, PTS_W)], yslab)
    pltpu.sync_copy(zs_hbm.at[pl.ds(wbase, PTS_W)], zslab)
    pltpu.sync_copy(batch_hbm.at[pl.ds(wbase, PTS_W)], bslab)

    def kbody(i, carry):
        x = xslab[pl.ds(i * L, L)]
        y = yslab[pl.ds(i * L, L)]
        z = zslab[pl.ds(i * L, L)]
        b = bslab[pl.ds(i * L, L)]
        key = (b * (OG * OG * OG) + (x >> 1) * (OG * OG)
               + (y >> 1) * OG + (z >> 1))
        kslab[pl.ds(i * L, L)] = key
        return carry

    lax.fori_loop(0, PTS_W // L, kbody, jnp.int32(0))

    sel = [(selk0, selp0, fsem0), (selk1, selp1, fsem1)]
    cnts_py = []
    for o in range(NW):
        selk, selp, fsem = sel[o % 2]
        if o >= 2:
            prev_cnt = cnts_py[o - 2]
            nblk_prev = (prev_cnt + (BLK - 1)) // BLK

            def wbody(i, carry, _selk=selk, _selp=selp, _fsem=fsem):
                pltpu.make_async_copy(
                    bins_hbm.at[pl.ds(0, BLK)], _selk.at[pl.ds(0, BLK)],
                    _fsem).wait()
                pltpu.make_async_copy(
                    bins_hbm.at[pl.ds(0, BLK)], _selp.at[pl.ds(0, BLK)],
                    _fsem).wait()
                return carry

            lax.fori_loop(0, nblk_prev, wbody, jnp.int32(0))

        def sbody(v, cnt, _o=o, _selk=selk, _selp=selp):
            k = kslab[pl.ds(v * L, L)]
            m = (k >> 12) == _o
            pid = wbase + v * L + lane
            plsc.store_compressed(_selk.at[pl.ds(cnt, L)], k, mask=m)
            plsc.store_compressed(_selp.at[pl.ds(cnt, L)], pid, mask=m)
            return cnt + jnp.sum(m.astype(jnp.int32))

        cnt = lax.fori_loop(0, PTS_W // L, sbody, jnp.int32(0))
        cnts_py.append(cnt)

        base_off = (wid * NW + o) * RCAP2
        nblk = (cnt + (BLK - 1)) // BLK

        def fbody(blk, carry, _selk=selk, _selp=selp, _fsem=fsem,
                  _base=base_off):
            off = _base + blk * BLKW
            pltpu.async_copy(_selk.at[pl.ds(blk * BLK, BLK)],
                             bins_hbm.at[pl.ds(off, BLK)], _fsem)
            pltpu.async_copy(_selp.at[pl.ds(blk * BLK, BLK)],
                             bins_hbm.at[pl.ds(off + BLK, BLK)], _fsem)
            return carry

        lax.fori_loop(0, nblk, fbody, jnp.int32(0))

    for o in (NW - 2, NW - 1):
        selk, selp, fsem = sel[o % 2]
        nblk_prev = (cnts_py[o] + (BLK - 1)) // BLK

        def wbody(i, carry, _selk=selk, _selp=selp, _fsem=fsem):
            pltpu.make_async_copy(bins_hbm.at[pl.ds(0, BLK)],
                                  _selk.at[pl.ds(0, BLK)], _fsem).wait()
            pltpu.make_async_copy(bins_hbm.at[pl.ds(0, BLK)],
                                  _selp.at[pl.ds(0, BLK)], _fsem).wait()
            return carry

        lax.fori_loop(0, nblk_prev, wbody, jnp.int32(0))

    v0 = jnp.zeros((L,), jnp.int32)
    v1 = jnp.zeros((L,), jnp.int32)
    for o in range(L):
        v0 = jnp.where(lane == o, cnts_py[o], v0)
        v1 = jnp.where(lane == o, cnts_py[L + o], v1)
    cbuf[pl.ds(0, L)] = v0
    cbuf[pl.ds(L, L)] = v1
    pltpu.sync_copy(cbuf, counts_hbm.at[pl.ds(wid * NW, NW)])


@functools.partial(
    pl.kernel,
    out_type=jax.ShapeDtypeStruct((NUM_SEGMENTS * C,), jnp.float32),
    mesh=_mesh,
    compiler_params=_params,
    scratch_types=[
        pltpu.VMEM((NW * NW + L,), jnp.int32),
        pltpu.VMEM((BMAX + L,), jnp.int32),
        pltpu.VMEM((BMAX + L,), jnp.int32),
        pltpu.VMEM((SCAP * BLKW,), jnp.int32),
        pltpu.VMEM((LCAP + 2 * L,), jnp.int32),
        pltpu.VMEM((LCAP + 2 * L,), jnp.int32),
        pltpu.VMEM((G, L, C), jnp.float32),
        pltpu.VMEM(((CHUNK + 1) * C,), jnp.float32),
        pltpu.SemaphoreType.DMA,
        pltpu.SemaphoreType.DMA,
    ],
)
def _pool_kernel(feats_hbm, bins_hbm, counts_hbm, out_hbm,
                 cvm, blkoff, blkval, stage, soff, spid, rowbuf, acc,
                 ksem, gsem):
    wid = lax.axis_index("s") * NC + lax.axis_index("c")
    lane = lax.iota(jnp.int32, L)

    pltpu.sync_copy(counts_hbm, cvm.at[pl.ds(0, NW * NW)])

    nb_total = jnp.int32(0)
    for wp in range(NW):
        c = cvm[pl.ds(wp * NW + wid, L)][0]
        nblk = (c + (BLK - 1)) // BLK
        src0 = (wp * NW + wid) * RCAP2

        def put(blk, pos, _src0=src0, _c=c):
            blkoff[pl.ds(pos, L)] = jnp.zeros((L,), jnp.int32) + (_src0 + blk * BLKW)
            blkval[pl.ds(pos, L)] = jnp.zeros((L,), jnp.int32) + jnp.minimum(_c - blk * BLK, BLK)
            return pos + 1

        nb_total = lax.fori_loop(0, nblk, put, nb_total)

    resident = nb_total <= SCAP

    def stream_blocks(start, nb_s):
        def fire(i, c2):
            off = pl.multiple_of(blkoff[pl.ds(start + i, L)][0], BLKW)
            pltpu.async_copy(bins_hbm.at[pl.ds(off, BLKW)],
                             stage.at[pl.ds(i * BLKW, BLKW)], ksem)
            return c2

        lax.fori_loop(0, nb_s, fire, jnp.int32(0))

        def drk(i, c2):
            pltpu.make_async_copy(bins_hbm.at[pl.ds(0, BLKW)],
                                  stage.at[pl.ds(0, BLKW)], ksem).wait()
            return c2

        lax.fori_loop(0, nb_s, drk, jnp.int32(0))

    @pl.when(resident)
    def _():
        stream_blocks(jnp.int32(0), nb_total)

    def drain(cnt):
        soff[pl.ds(cnt, L)] = jnp.full((L,), DUMMY_OFF, jnp.int32)
        spid[pl.ds(cnt, L)] = jnp.zeros((L,), jnp.int32)
        nb = (cnt + (L - 1)) // L

        def super_g(sg, carry):
            gstart = sg * G
            ng = jnp.minimum(nb - gstart, G)

            def fire2(i, c2):
                idxv = spid[pl.ds((gstart + i) * L, L)]
                pltpu.async_copy(feats_hbm.at[idxv], rowbuf.at[i], gsem)
                return c2

            lax.fori_loop(0, ng, fire2, jnp.int32(0))

            def drng(i, c2):
                idxv = spid[pl.ds((gstart + i) * L, L)]
                pltpu.make_async_copy(feats_hbm.at[idxv], rowbuf.at[i],
                                      gsem).wait()
                return c2

            lax.fori_loop(0, ng, drng, jnp.int32(0))

            def gbody(i, c2):
                def pbody(ii, c3):
                    off = soff[pl.ds((gstart + i) * L + ii, L)][0]
                    for jj in range(C // L):
                        a = acc[pl.ds(off + jj * L, L)]
                        r = rowbuf[i, ii, pl.ds(jj * L, L)]
                        acc[pl.ds(off + jj * L, L)] = jnp.maximum(a, r)
                    return c3

                lax.fori_loop(0, L, pbody, jnp.int32(0))
                return c2

            lax.fori_loop(0, ng, gbody, jnp.int32(0))
            return carry

        nsg = (nb + (G - 1)) // G
        lax.fori_loop(0, nsg, super_g, jnp.int32(0))

    def run_pass(p, carry):
        base_row = wid * OWN_ROWS + p * CHUNK

        neg = jnp.full((L,), NEG, jnp.float32)

        def ibody(v, c):
            for u in range(16):
                acc[pl.ds(v * (16 * L) + u * L, L)] = neg
            return c

        lax.fori_loop(0, CHUNK * C // (16 * L), ibody, jnp.int32(0))

        def super_body(s, cnt):
            start = s * SCAP
            nb_s = jnp.minimum(nb_total - start, SCAP)

            @pl.when(jnp.logical_not(resident))
            def _():
                stream_blocks(start, nb_s)

            def block_body(i, cnt2):
                val = blkval[pl.ds(start + i, L)][0]
                sbase = i * BLKW

                def scan_step(j, cnt3):
                    k = stage[pl.ds(sbase + j * L, L)]
                    pidv = stage[pl.ds(sbase + BLK + j * L, L)]
                    rel = k - base_row
                    m = ((j * L + lane) < val) & (rel >= 0) & (rel < CHUNK)
                    plsc.store_compressed(soff.at[pl.ds(cnt3, L)], rel * C,
                                          mask=m)
                    plsc.store_compressed(spid.at[pl.ds(cnt3, L)], pidv,
                                          mask=m)
                    return cnt3 + jnp.sum(m.astype(jnp.int32))

                for j in range(BLK // L):
                    cnt2 = scan_step(j, cnt2)

                def do_drain(c):
                    drain(c)
                    return jnp.int32(0)

                return lax.cond(cnt2 >= DRAIN_T, do_drain, lambda c: c, cnt2)

            return lax.fori_loop(0, nb_s, block_body, cnt)

        nsuper = (nb_total + (SCAP - 1)) // SCAP
        cnt = lax.fori_loop(0, nsuper, super_body, jnp.int32(0))
        drain(cnt)

        def fbody(v, c):
            for u in range(8):
                a = acc[pl.ds(v * (8 * L) + u * L, L)]
                acc[pl.ds(v * (8 * L) + u * L, L)] = jnp.where(
                    a == NEG, jnp.float32(0.0), a)
            return c

        lax.fori_loop(0, CHUNK * C // (8 * L), fbody, jnp.int32(0))
        pltpu.sync_copy(acc.at[pl.ds(0, CHUNK * C)],
                        out_hbm.at[pl.ds(base_row * C, CHUNK * C)])
        return carry

    lax.fori_loop(0, PASSES, run_pass, jnp.int32(0))


def kernel(feats, coords, batch_idx):
    zpad = jnp.zeros((PAD,), jnp.int32)
    xs = jnp.concatenate([coords[:, 0], zpad])
    ys = jnp.concatenate([coords[:, 1], zpad])
    zs = jnp.concatenate([coords[:, 2], zpad])
    batch_flat = jnp.concatenate(
        [batch_idx.reshape(-1).astype(jnp.int32),
         jnp.full((PAD,), BATCH, jnp.int32)])
    bins, counts = _bin_kernel(xs, ys, zs, batch_flat)
    out = _pool_kernel(feats, bins, counts)
    return out.reshape(NUM_SEGMENTS, C)


# speedup vs baseline: 4.0757x; 1.1326x over previous
import functools

import jax
import jax.numpy as jnp
from jax import lax
from jax.experimental import pallas as pl
from jax.experimental.pallas import tpu as pltpu
from jax.experimental.pallas import tpu_sc as plsc

GRID = 64
STRIDE = 2
OG = GRID // STRIDE
BATCH = 4
N = 100000
C = 256
NUM_SEGMENTS = BATCH * OG * OG * OG

NC, NS, L = 2, 16, 16
NW = NC * NS
PTS_W = 3136
NP = PTS_W * NW
PAD = NP - N

OWN_ROWS = NUM_SEGMENTS // NW
CHUNK = 256
PASSES = OWN_ROWS // CHUNK

BLK = 128
BLKW = 2 * BLK
NBLK_W = PTS_W // BLK
RCAP = ((PTS_W + BLK - 1) // BLK) * BLK
RCAP2 = 2 * RCAP
SELCAP = RCAP + 2 * L
BMAX = NW * (RCAP // BLK) + L
SCAP = 64
DRAIN_T = 2048
LCAP = DRAIN_T + BLK + 2 * L
G2 = 4
DUMMY_OFF = CHUNK * C
NEG = float("-inf")

_mesh = plsc.VectorSubcoreMesh(core_axis_name="c", subcore_axis_name="s")
_params = pltpu.CompilerParams(needs_layout_passes=False)


@functools.partial(
    pl.kernel,
    out_type=(jax.ShapeDtypeStruct((NW * NW * RCAP2,), jnp.int32),
              jax.ShapeDtypeStruct((NW * NW,), jnp.int32)),
    mesh=_mesh,
    compiler_params=_params,
    scratch_types=[
        pltpu.VMEM((PTS_W,), jnp.int32),
        pltpu.VMEM((PTS_W,), jnp.int32),
        pltpu.VMEM((PTS_W,), jnp.int32),
        pltpu.VMEM((PTS_W,), jnp.int32),
        pltpu.VMEM((PTS_W,), jnp.int32),
        pltpu.SemaphoreType.DMA,
        pltpu.VMEM((SELCAP,), jnp.int32),
        pltpu.VMEM((SELCAP,), jnp.int32),
        pltpu.VMEM((SELCAP,), jnp.int32),
        pltpu.VMEM((SELCAP,), jnp.int32),
        pltpu.VMEM((32,), jnp.int32),
        pltpu.SemaphoreType.DMA,
        pltpu.SemaphoreType.DMA,
    ],
)
def _bin_kernel(xs_hbm, ys_hbm, zs_hbm, batch_hbm, bins_hbm, counts_hbm,
                xslab, yslab, zslab, bslab, kslab, ssem,
                selk0, selp0, selk1, selp1, cbuf, fsem0, fsem1):
    wid = lax.axis_index("s") * NC + lax.axis_index("c")
    wbase = wid * PTS_W
    lane = lax.iota(jnp.int32, L)

    pltpu.async_copy(xs_hbm.at[pl.ds(wbase, PTS_W)], xslab, ssem)
    pltpu.async_copy(ys_hbm.at[pl.ds(wbase, PTS_W)], yslab, ssem)
    pltpu.async_copy(zs_hbm.at[pl.ds(wbase, PTS_W)], zslab, ssem)
    pltpu.async_copy(batch_hbm.at[pl.ds(wbase, PTS_W)], bslab, ssem)
    pltpu.make_async_copy(xs_hbm.at[pl.ds(wbase, PTS_W)], xslab, ssem).wait()
    pltpu.make_async_copy(ys_hbm.at[pl.ds(wbase, PTS_W)], yslab, ssem).wait()
    pltpu.make_async_copy(zs_hbm.at[pl.ds(wbase, PTS_W)], zslab, ssem).wait()
    pltpu.make_async_copy(batch_hbm.at[pl.ds(wbase, PTS_W)], bslab,
                          ssem).wait()

    def kbody(i, carry):
        x = xslab[pl.ds(i * L, L)]
        y = yslab[pl.ds(i * L, L)]
        z = zslab[pl.ds(i * L, L)]
        b = bslab[pl.ds(i * L, L)]
        key = (b * (OG * OG * OG) + (x >> 1) * (OG * OG)
               + (y >> 1) * OG + (z >> 1))
        kslab[pl.ds(i * L, L)] = key
        return carry

    lax.fori_loop(0, PTS_W // L, kbody, jnp.int32(0))

    sel = [(selk0, selp0, fsem0), (selk1, selp1, fsem1)]
    cnts_py = []
    for o in range(NW):
        selk, selp, fsem = sel[o % 2]
        if o >= 2:
            prev_cnt = cnts_py[o - 2]
            nblk_prev = (prev_cnt + (BLK - 1)) // BLK

            def wbody(i, carry, _selk=selk, _selp=selp, _fsem=fsem):
                pltpu.make_async_copy(
                    bins_hbm.at[pl.ds(0, BLK)], _selk.at[pl.ds(0, BLK)],
                    _fsem).wait()
                pltpu.make_async_copy(
                    bins_hbm.at[pl.ds(0, BLK)], _selp.at[pl.ds(0, BLK)],
                    _fsem).wait()
                return carry

            lax.fori_loop(0, nblk_prev, wbody, jnp.int32(0))

        def sbody(v, cnt, _o=o, _selk=selk, _selp=selp):
            k = kslab[pl.ds(v * L, L)]
            m = (k >> 12) == _o
            pid = wbase + v * L + lane
            plsc.store_compressed(_selk.at[pl.ds(cnt, L)], k, mask=m)
            plsc.store_compressed(_selp.at[pl.ds(cnt, L)], pid, mask=m)
            return cnt + jnp.sum(m.astype(jnp.int32))

        cnt = lax.fori_loop(0, PTS_W // L, sbody, jnp.int32(0))
        cnts_py.append(cnt)

        base_off = (wid * NW + o) * RCAP2
        nblk = (cnt + (BLK - 1)) // BLK

        def fbody(blk, carry, _selk=selk, _selp=selp, _fsem=fsem,
                  _base=base_off):
            off = _base + blk * BLKW
            pltpu.async_copy(_selk.at[pl.ds(blk * BLK, BLK)],
                             bins_hbm.at[pl.ds(off, BLK)], _fsem)
            pltpu.async_copy(_selp.at[pl.ds(blk * BLK, BLK)],
                             bins_hbm.at[pl.ds(off + BLK, BLK)], _fsem)
            return carry

        lax.fori_loop(0, nblk, fbody, jnp.int32(0))

    for o in (NW - 2, NW - 1):
        selk, selp, fsem = sel[o % 2]
        nblk_prev = (cnts_py[o] + (BLK - 1)) // BLK

        def wbody(i, carry, _selk=selk, _selp=selp, _fsem=fsem):
            pltpu.make_async_copy(bins_hbm.at[pl.ds(0, BLK)],
                                  _selk.at[pl.ds(0, BLK)], _fsem).wait()
            pltpu.make_async_copy(bins_hbm.at[pl.ds(0, BLK)],
                                  _selp.at[pl.ds(0, BLK)], _fsem).wait()
            return carry

        lax.fori_loop(0, nblk_prev, wbody, jnp.int32(0))

    v0 = jnp.zeros((L,), jnp.int32)
    v1 = jnp.zeros((L,), jnp.int32)
    for o in range(L):
        v0 = jnp.where(lane == o, cnts_py[o], v0)
        v1 = jnp.where(lane == o, cnts_py[L + o], v1)
    cbuf[pl.ds(0, L)] = v0
    cbuf[pl.ds(L, L)] = v1
    pltpu.sync_copy(cbuf, counts_hbm.at[pl.ds(wid * NW, NW)])


@functools.partial(
    pl.kernel,
    out_type=jax.ShapeDtypeStruct((NUM_SEGMENTS * C,), jnp.float32),
    mesh=_mesh,
    compiler_params=_params,
    scratch_types=[
        pltpu.VMEM((NW * NW + L,), jnp.int32),
        pltpu.VMEM((BMAX + L,), jnp.int32),
        pltpu.VMEM((BMAX + L,), jnp.int32),
        pltpu.VMEM((SCAP * BLKW,), jnp.int32),
        pltpu.VMEM((LCAP + 2 * L,), jnp.int32),
        pltpu.VMEM((LCAP + 2 * L,), jnp.int32),
        pltpu.VMEM((2, G2, L, C), jnp.float32),
        pltpu.VMEM(((CHUNK + 1) * C,), jnp.float32),
        pltpu.SemaphoreType.DMA,
        pltpu.SemaphoreType.DMA,
        pltpu.SemaphoreType.DMA,
        pltpu.SemaphoreType.DMA,
    ],
)
def _pool_kernel(feats_hbm, bins_hbm, counts_hbm, out_hbm,
                 cvm, blkoff, blkval, stage, soff, spid, rowbuf, acc,
                 ksem, gsem0, gsem1, wsem):
    wid = lax.axis_index("s") * NC + lax.axis_index("c")
    lane = lax.iota(jnp.int32, L)

    pltpu.sync_copy(counts_hbm, cvm.at[pl.ds(0, NW * NW)])

    nb_total = jnp.int32(0)
    for wp in range(NW):
        c = cvm[pl.ds(wp * NW + wid, L)][0]
        nblk = (c + (BLK - 1)) // BLK
        src0 = (wp * NW + wid) * RCAP2

        def put(blk, pos, _src0=src0, _c=c):
            blkoff[pl.ds(pos, L)] = jnp.zeros((L,), jnp.int32) + (_src0 + blk * BLKW)
            blkval[pl.ds(pos, L)] = jnp.zeros((L,), jnp.int32) + jnp.minimum(_c - blk * BLK, BLK)
            return pos + 1

        nb_total = lax.fori_loop(0, nblk, put, nb_total)

    resident = nb_total <= SCAP

    def stream_blocks(start, nb_s):
        def fire(i, c2):
            off = pl.multiple_of(blkoff[pl.ds(start + i, L)][0], BLKW)
            pltpu.async_copy(bins_hbm.at[pl.ds(off, BLKW)],
                             stage.at[pl.ds(i * BLKW, BLKW)], ksem)
            return c2

        lax.fori_loop(0, nb_s, fire, jnp.int32(0))

        def drk(i, c2):
            pltpu.make_async_copy(bins_hbm.at[pl.ds(0, BLKW)],
                                  stage.at[pl.ds(0, BLKW)], ksem).wait()
            return c2

        lax.fori_loop(0, nb_s, drk, jnp.int32(0))

    @pl.when(resident)
    def _():
        stream_blocks(jnp.int32(0), nb_total)

    def drain(cnt):
        soff[pl.ds(cnt, L)] = jnp.full((L,), DUMMY_OFF, jnp.int32)
        spid[pl.ds(cnt, L)] = jnp.zeros((L,), jnp.int32)
        nb = (cnt + (L - 1)) // L
        nsg = (nb + (G2 - 1)) // G2

        def fire_batch(sg, par, sem):
            gstart = sg * G2
            ng = jnp.minimum(nb - gstart, G2)

            def fire(i, c2):
                idxv = spid[pl.ds((gstart + i) * L, L)]
                pltpu.async_copy(feats_hbm.at[idxv], rowbuf.at[par, i], sem)
                return c2

            lax.fori_loop(0, ng, fire, jnp.int32(0))

        def proc_batch(sg, par, sem):
            gstart = sg * G2
            ng = jnp.minimum(nb - gstart, G2)

            def drng(i, c2):
                idxv = spid[pl.ds((gstart + i) * L, L)]
                pltpu.make_async_copy(feats_hbm.at[idxv], rowbuf.at[par, i],
                                      sem).wait()
                return c2

            lax.fori_loop(0, ng, drng, jnp.int32(0))

            def gbody(i, c2):
                def pbody(ii, c3):
                    off = soff[pl.ds((gstart + i) * L + ii, L)][0]
                    for jj in range(C // L):
                        a = acc[pl.ds(off + jj * L, L)]
                        r = rowbuf[par, i, ii, pl.ds(jj * L, L)]
                        acc[pl.ds(off + jj * L, L)] = jnp.maximum(a, r)
                    return c3

                lax.fori_loop(0, L, pbody, jnp.int32(0))
                return c2

            lax.fori_loop(0, ng, gbody, jnp.int32(0))

        fire_batch(jnp.int32(0), 0, gsem0)

        def duo(i2, carry):
            sg0 = 2 * i2
            sg1 = sg0 + 1

            @pl.when(sg1 < nsg)
            def _():
                fire_batch(sg1, 1, gsem1)

            proc_batch(sg0, 0, gsem0)

            @pl.when(sg1 + 1 < nsg)
            def _():
                fire_batch(sg1 + 1, 0, gsem0)

            @pl.when(sg1 < nsg)
            def _():
                proc_batch(sg1, 1, gsem1)

            return carry

        lax.fori_loop(0, (nsg + 1) // 2, duo, jnp.int32(0))

    neg = jnp.full((L,), NEG, jnp.float32)

    def run_pass(p, carry):
        base_row = wid * OWN_ROWS + p * CHUNK

        def ensure_acc(ready):
            def do(r):
                @pl.when(p > 0)
                def _():
                    pltpu.make_async_copy(
                        acc.at[pl.ds(0, CHUNK * C)],
                        out_hbm.at[pl.ds(0, CHUNK * C)], wsem).wait()

                def ibody(v, c):
                    for u in range(16):
                        acc[pl.ds(v * (16 * L) + u * L, L)] = neg
                    return c

                lax.fori_loop(0, CHUNK * C // (16 * L), ibody, jnp.int32(0))
                return jnp.int32(1)

            return lax.cond(ready == 0, do, lambda r: r, ready)

        def super_body(s, cr):
            cnt, ready = cr
            start = s * SCAP
            nb_s = jnp.minimum(nb_total - start, SCAP)

            @pl.when(jnp.logical_not(resident))
            def _():
                stream_blocks(start, nb_s)

            def block_body(i, cr2):
                cnt2, ready2 = cr2
                val = blkval[pl.ds(start + i, L)][0]
                sbase = i * BLKW

                def scan_step(j, cnt3):
                    k = stage[pl.ds(sbase + j * L, L)]
                    pidv = stage[pl.ds(sbase + BLK + j * L, L)]
                    rel = k - base_row
                    m = ((j * L + lane) < val) & (rel >= 0) & (rel < CHUNK)
                    plsc.store_compressed(soff.at[pl.ds(cnt3, L)], rel * C,
                                          mask=m)
                    plsc.store_compressed(spid.at[pl.ds(cnt3, L)], pidv,
                                          mask=m)
                    return cnt3 + jnp.sum(m.astype(jnp.int32))

                for j in range(BLK // L):
                    cnt2 = scan_step(j, cnt2)

                def do_drain(cr3):
                    c3, r3 = cr3
                    r3 = ensure_acc(r3)
                    drain(c3)
                    return jnp.int32(0), r3

                return lax.cond(cnt2 >= DRAIN_T, do_drain,
                                lambda cr3: cr3, (cnt2, ready2))

            return lax.fori_loop(0, nb_s, block_body, (cnt, ready))

        nsuper = (nb_total + (SCAP - 1)) // SCAP
        cnt, ready = lax.fori_loop(0, nsuper, super_body,
                                   (jnp.int32(0), jnp.int32(0)))
        ready = ensure_acc(ready)
        drain(cnt)

        def fbody(v, c):
            for u in range(8):
                a = acc[pl.ds(v * (8 * L) + u * L, L)]
                acc[pl.ds(v * (8 * L) + u * L, L)] = jnp.where(
                    a == NEG, jnp.float32(0.0), a)
            return c

        lax.fori_loop(0, CHUNK * C // (8 * L), fbody, jnp.int32(0))
        pltpu.async_copy(acc.at[pl.ds(0, CHUNK * C)],
                         out_hbm.at[pl.ds(base_row * C, CHUNK * C)], wsem)
        return carry

    lax.fori_loop(0, PASSES, run_pass, jnp.int32(0))
    pltpu.make_async_copy(acc.at[pl.ds(0, CHUNK * C)],
                          out_hbm.at[pl.ds(0, CHUNK * C)], wsem).wait()


def kernel(feats, coords, batch_idx):
    zpad = jnp.zeros((PAD,), jnp.int32)
    xs = jnp.concatenate([coords[:, 0], zpad])
    ys = jnp.concatenate([coords[:, 1], zpad])
    zs = jnp.concatenate([coords[:, 2], zpad])
    batch_flat = jnp.concatenate(
        [batch_idx.reshape(-1).astype(jnp.int32),
         jnp.full((PAD,), BATCH, jnp.int32)])
    bins, counts = _bin_kernel(xs, ys, zs, batch_flat)
    out = _pool_kernel(feats, bins, counts)
    return out.reshape(NUM_SEGMENTS, C)


# speedup vs baseline: 4.0814x; 1.0014x over previous
import functools

import jax
import jax.numpy as jnp
from jax import lax
from jax.experimental import pallas as pl
from jax.experimental.pallas import tpu as pltpu
from jax.experimental.pallas import tpu_sc as plsc

GRID = 64
STRIDE = 2
OG = GRID // STRIDE
BATCH = 4
N = 100000
C = 256
NUM_SEGMENTS = BATCH * OG * OG * OG

NC, NS, L = 2, 16, 16
NW = NC * NS
PTS_W = 3136
NP = PTS_W * NW
PAD = NP - N

OWN_ROWS = NUM_SEGMENTS // NW
CHUNK = 256
PASSES = OWN_ROWS // CHUNK

BLK = 128
BLKW = 2 * BLK
NBLK_W = PTS_W // BLK
RCAP = ((PTS_W + BLK - 1) // BLK) * BLK
RCAP2 = 2 * RCAP
SELCAP = RCAP + 2 * L
BMAX = NW * (RCAP // BLK) + L
SCAP = 64
DRAIN_T = 2048
LCAP = DRAIN_T + BLK + 2 * L
G2 = 4
DUMMY_OFF = CHUNK * C
NEG = float("-inf")

_mesh = plsc.VectorSubcoreMesh(core_axis_name="c", subcore_axis_name="s")
_params = pltpu.CompilerParams(needs_layout_passes=False)


@functools.partial(
    pl.kernel,
    out_type=(jax.ShapeDtypeStruct((NW * NW * RCAP2,), jnp.int32),
              jax.ShapeDtypeStruct((NW * NW,), jnp.int32)),
    mesh=_mesh,
    compiler_params=_params,
    scratch_types=[
        pltpu.VMEM((PTS_W,), jnp.int32),
        pltpu.VMEM((PTS_W,), jnp.int32),
        pltpu.VMEM((PTS_W,), jnp.int32),
        pltpu.VMEM((PTS_W,), jnp.int32),
        pltpu.VMEM((PTS_W,), jnp.int32),
        pltpu.SemaphoreType.DMA,
        pltpu.VMEM((SELCAP,), jnp.int32),
        pltpu.VMEM((SELCAP,), jnp.int32),
        pltpu.VMEM((SELCAP,), jnp.int32),
        pltpu.VMEM((SELCAP,), jnp.int32),
        pltpu.VMEM((32,), jnp.int32),
        pltpu.SemaphoreType.DMA,
        pltpu.SemaphoreType.DMA,
    ],
)
def _bin_kernel(xs_hbm, ys_hbm, zs_hbm, batch_hbm, bins_hbm, counts_hbm,
                xslab, yslab, zslab, bslab, kslab, ssem,
                selk0, selp0, selk1, selp1, cbuf, fsem0, fsem1):
    wid = lax.axis_index("s") * NC + lax.axis_index("c")
    wbase = wid * PTS_W
    lane = lax.iota(jnp.int32, L)

    pltpu.async_copy(xs_hbm.at[pl.ds(wbase, PTS_W)], xslab, ssem)
    pltpu.async_copy(ys_hbm.at[pl.ds(wbase, PTS_W)], yslab, ssem)
    pltpu.async_copy(zs_hbm.at[pl.ds(wbase, PTS_W)], zslab, ssem)
    pltpu.async_copy(batch_hbm.at[pl.ds(wbase, PTS_W)], bslab, ssem)
    pltpu.make_async_copy(xs_hbm.at[pl.ds(wbase, PTS_W)], xslab, ssem).wait()
    pltpu.make_async_copy(ys_hbm.at[pl.ds(wbase, PTS_W)], yslab, ssem).wait()
    pltpu.make_async_copy(zs_hbm.at[pl.ds(wbase, PTS_W)], zslab, ssem).wait()
    pltpu.make_async_copy(batch_hbm.at[pl.ds(wbase, PTS_W)], bslab,
                          ssem).wait()

    def kbody(i, carry):
        for u in range(7):
            q = (7 * i + u) * L
            x = xslab[pl.ds(q, L)]
            y = yslab[pl.ds(q, L)]
            z = zslab[pl.ds(q, L)]
            b = bslab[pl.ds(q, L)]
            key = (b * (OG * OG * OG) + (x >> 1) * (OG * OG)
                   + (y >> 1) * OG + (z >> 1))
            kslab[pl.ds(q, L)] = key
        return carry

    lax.fori_loop(0, PTS_W // (7 * L), kbody, jnp.int32(0))

    sel = [(selk0, selp0, fsem0), (selk1, selp1, fsem1)]
    cnts_py = []
    for o in range(NW):
        selk, selp, fsem = sel[o % 2]
        if o >= 2:
            prev_cnt = cnts_py[o - 2]
            nblk_prev = (prev_cnt + (BLK - 1)) // BLK

            def wbody(i, carry, _selk=selk, _selp=selp, _fsem=fsem):
                pltpu.make_async_copy(
                    bins_hbm.at[pl.ds(0, BLK)], _selk.at[pl.ds(0, BLK)],
                    _fsem).wait()
                pltpu.make_async_copy(
                    bins_hbm.at[pl.ds(0, BLK)], _selp.at[pl.ds(0, BLK)],
                    _fsem).wait()
                return carry

            lax.fori_loop(0, nblk_prev, wbody, jnp.int32(0))

        def sbody(v, cnt, _o=o, _selk=selk, _selp=selp):
            for u in range(2):
                k = kslab[pl.ds((2 * v + u) * L, L)]
                m = (k >> 12) == _o
                pid = wbase + (2 * v + u) * L + lane
                plsc.store_compressed(_selk.at[pl.ds(cnt, L)], k, mask=m)
                plsc.store_compressed(_selp.at[pl.ds(cnt, L)], pid, mask=m)
                cnt = cnt + jnp.sum(m.astype(jnp.int32))
            return cnt

        cnt = lax.fori_loop(0, PTS_W // (2 * L), sbody, jnp.int32(0))
        cnts_py.append(cnt)

        base_off = (wid * NW + o) * RCAP2
        nblk = (cnt + (BLK - 1)) // BLK

        def fbody(blk, carry, _selk=selk, _selp=selp, _fsem=fsem,
                  _base=base_off):
            off = _base + blk * BLKW
            pltpu.async_copy(_selk.at[pl.ds(blk * BLK, BLK)],
                             bins_hbm.at[pl.ds(off, BLK)], _fsem)
            pltpu.async_copy(_selp.at[pl.ds(blk * BLK, BLK)],
                             bins_hbm.at[pl.ds(off + BLK, BLK)], _fsem)
            return carry

        lax.fori_loop(0, nblk, fbody, jnp.int32(0))

    for o in (NW - 2, NW - 1):
        selk, selp, fsem = sel[o % 2]
        nblk_prev = (cnts_py[o] + (BLK - 1)) // BLK

        def wbody(i, carry, _selk=selk, _selp=selp, _fsem=fsem):
            pltpu.make_async_copy(bins_hbm.at[pl.ds(0, BLK)],
                                  _selk.at[pl.ds(0, BLK)], _fsem).wait()
            pltpu.make_async_copy(bins_hbm.at[pl.ds(0, BLK)],
                                  _selp.at[pl.ds(0, BLK)], _fsem).wait()
            return carry

        lax.fori_loop(0, nblk_prev, wbody, jnp.int32(0))

    v0 = jnp.zeros((L,), jnp.int32)
    v1 = jnp.zeros((L,), jnp.int32)
    for o in range(L):
        v0 = jnp.where(lane == o, cnts_py[o], v0)
        v1 = jnp.where(lane == o, cnts_py[L + o], v1)
    cbuf[pl.ds(0, L)] = v0
    cbuf[pl.ds(L, L)] = v1
    pltpu.sync_copy(cbuf, counts_hbm.at[pl.ds(wid * NW, NW)])


@functools.partial(
    pl.kernel,
    out_type=jax.ShapeDtypeStruct((NUM_SEGMENTS * C,), jnp.float32),
    mesh=_mesh,
    compiler_params=_params,
    scratch_types=[
        pltpu.VMEM((NW * NW + L,), jnp.int32),
        pltpu.VMEM((BMAX + L,), jnp.int32),
        pltpu.VMEM((BMAX + L,), jnp.int32),
        pltpu.VMEM((SCAP * BLKW,), jnp.int32),
        pltpu.VMEM((LCAP + 2 * L,), jnp.int32),
        pltpu.VMEM((LCAP + 2 * L,), jnp.int32),
        pltpu.VMEM((2, G2, L, C), jnp.float32),
        pltpu.VMEM(((CHUNK + 1) * C,), jnp.float32),
        pltpu.SemaphoreType.DMA,
        pltpu.SemaphoreType.DMA,
        pltpu.SemaphoreType.DMA,
        pltpu.SemaphoreType.DMA,
    ],
)
def _pool_kernel(feats_hbm, bins_hbm, counts_hbm, out_hbm,
                 cvm, blkoff, blkval, stage, soff, spid, rowbuf, acc,
                 ksem, gsem0, gsem1, wsem):
    wid = lax.axis_index("s") * NC + lax.axis_index("c")
    lane = lax.iota(jnp.int32, L)

    pltpu.sync_copy(counts_hbm, cvm.at[pl.ds(0, NW * NW)])

    nb_total = jnp.int32(0)
    for wp in range(NW):
        c = cvm[pl.ds(wp * NW + wid, L)][0]
        nblk = (c + (BLK - 1)) // BLK
        src0 = (wp * NW + wid) * RCAP2

        def put(blk, pos, _src0=src0, _c=c):
            blkoff[pl.ds(pos, L)] = jnp.zeros((L,), jnp.int32) + (_src0 + blk * BLKW)
            blkval[pl.ds(pos, L)] = jnp.zeros((L,), jnp.int32) + jnp.minimum(_c - blk * BLK, BLK)
            return pos + 1

        nb_total = lax.fori_loop(0, nblk, put, nb_total)

    resident = nb_total <= SCAP

    def stream_blocks(start, nb_s):
        def fire(i, c2):
            off = pl.multiple_of(blkoff[pl.ds(start + i, L)][0], BLKW)
            pltpu.async_copy(bins_hbm.at[pl.ds(off, BLKW)],
                             stage.at[pl.ds(i * BLKW, BLKW)], ksem)
            return c2

        lax.fori_loop(0, nb_s, fire, jnp.int32(0))

        def drk(i, c2):
            pltpu.make_async_copy(bins_hbm.at[pl.ds(0, BLKW)],
                                  stage.at[pl.ds(0, BLKW)], ksem).wait()
            return c2

        lax.fori_loop(0, nb_s, drk, jnp.int32(0))

    @pl.when(resident)
    def _():
        stream_blocks(jnp.int32(0), nb_total)

    def drain(cnt):
        soff[pl.ds(cnt, L)] = jnp.full((L,), DUMMY_OFF, jnp.int32)
        spid[pl.ds(cnt, L)] = jnp.zeros((L,), jnp.int32)
        nb = (cnt + (L - 1)) // L
        nsg = (nb + (G2 - 1)) // G2

        def fire_batch(sg, par, sem):
            gstart = sg * G2
            ng = jnp.minimum(nb - gstart, G2)

            def fire(i, c2):
                idxv = spid[pl.ds((gstart + i) * L, L)]
                pltpu.async_copy(feats_hbm.at[idxv], rowbuf.at[par, i], sem)
                return c2

            lax.fori_loop(0, ng, fire, jnp.int32(0))

        def proc_batch(sg, par, sem):
            gstart = sg * G2
            ng = jnp.minimum(nb - gstart, G2)

            def drng(i, c2):
                idxv = spid[pl.ds((gstart + i) * L, L)]
                pltpu.make_async_copy(feats_hbm.at[idxv], rowbuf.at[par, i],
                                      sem).wait()
                return c2

            lax.fori_loop(0, ng, drng, jnp.int32(0))

            def gbody(i, c2):
                offv = soff[pl.ds((gstart + i) * L, L)]
                for ii in range(L):
                    off = offv[ii]
                    for jj in range(C // L):
                        a = acc[pl.ds(off + jj * L, L)]
                        r = rowbuf[par, i, ii, pl.ds(jj * L, L)]
                        acc[pl.ds(off + jj * L, L)] = jnp.maximum(a, r)
                return c2

            lax.fori_loop(0, ng, gbody, jnp.int32(0))

        fire_batch(jnp.int32(0), 0, gsem0)

        def duo(i2, carry):
            sg0 = 2 * i2
            sg1 = sg0 + 1

            @pl.when(sg1 < nsg)
            def _():
                fire_batch(sg1, 1, gsem1)

            proc_batch(sg0, 0, gsem0)

            @pl.when(sg1 + 1 < nsg)
            def _():
                fire_batch(sg1 + 1, 0, gsem0)

            @pl.when(sg1 < nsg)
            def _():
                proc_batch(sg1, 1, gsem1)

            return carry

        lax.fori_loop(0, (nsg + 1) // 2, duo, jnp.int32(0))

    neg = jnp.full((L,), NEG, jnp.float32)

    def run_pass(p, carry):
        base_row = wid * OWN_ROWS + p * CHUNK

        def ensure_acc(ready):
            def do(r):
                @pl.when(p > 0)
                def _():
                    pltpu.make_async_copy(
                        acc.at[pl.ds(0, CHUNK * C)],
                        out_hbm.at[pl.ds(0, CHUNK * C)], wsem).wait()

                def ibody(v, c):
                    for u in range(16):
                        acc[pl.ds(v * (16 * L) + u * L, L)] = neg
                    return c

                lax.fori_loop(0, CHUNK * C // (16 * L), ibody, jnp.int32(0))
                return jnp.int32(1)

            return lax.cond(ready == 0, do, lambda r: r, ready)

        def super_body(s, cr):
            cnt, ready = cr
            start = s * SCAP
            nb_s = jnp.minimum(nb_total - start, SCAP)

            @pl.when(jnp.logical_not(resident))
            def _():
                stream_blocks(start, nb_s)

            def block_body(i, cr2):
                cnt2, ready2 = cr2
                val = blkval[pl.ds(start + i, L)][0]
                sbase = i * BLKW

                def scan_step(j, cnt3):
                    k = stage[pl.ds(sbase + j * L, L)]
                    pidv = stage[pl.ds(sbase + BLK + j * L, L)]
                    rel = k - base_row
                    m = ((j * L + lane) < val) & (rel >= 0) & (rel < CHUNK)
                    plsc.store_compressed(soff.at[pl.ds(cnt3, L)], rel * C,
                                          mask=m)
                    plsc.store_compressed(spid.at[pl.ds(cnt3, L)], pidv,
                                          mask=m)
                    return cnt3 + jnp.sum(m.astype(jnp.int32))

                for j in range(BLK // L):
                    cnt2 = scan_step(j, cnt2)

                def do_drain(cr3):
                    c3, r3 = cr3
                    r3 = ensure_acc(r3)
                    drain(c3)
                    return jnp.int32(0), r3

                return lax.cond(cnt2 >= DRAIN_T, do_drain,
                                lambda cr3: cr3, (cnt2, ready2))

            return lax.fori_loop(0, nb_s, block_body, (cnt, ready))

        nsuper = (nb_total + (SCAP - 1)) // SCAP
        cnt, ready = lax.fori_loop(0, nsuper, super_body,
                                   (jnp.int32(0), jnp.int32(0)))
        ready = ensure_acc(ready)
        drain(cnt)

        def fbody(v, c):
            for u in range(8):
                a = acc[pl.ds(v * (8 * L) + u * L, L)]
                acc[pl.ds(v * (8 * L) + u * L, L)] = jnp.where(
                    a == NEG, jnp.float32(0.0), a)
            return c

        lax.fori_loop(0, CHUNK * C // (8 * L), fbody, jnp.int32(0))
        pltpu.async_copy(acc.at[pl.ds(0, CHUNK * C)],
                         out_hbm.at[pl.ds(base_row * C, CHUNK * C)], wsem)
        return carry

    lax.fori_loop(0, PASSES, run_pass, jnp.int32(0))
    pltpu.make_async_copy(acc.at[pl.ds(0, CHUNK * C)],
                          out_hbm.at[pl.ds(0, CHUNK * C)], wsem).wait()


def kernel(feats, coords, batch_idx):
    zpad = jnp.zeros((PAD,), jnp.int32)
    xs = jnp.concatenate([coords[:, 0], zpad])
    ys = jnp.concatenate([coords[:, 1], zpad])
    zs = jnp.concatenate([coords[:, 2], zpad])
    batch_flat = jnp.concatenate(
        [batch_idx.reshape(-1).astype(jnp.int32),
         jnp.full((PAD,), BATCH, jnp.int32)])
    bins, counts = _bin_kernel(xs, ys, zs, batch_flat)
    out = _pool_kernel(feats, bins, counts)
    return out.reshape(NUM_SEGMENTS, C)
